# Initial kernel scaffold; baseline (speedup 1.0000x reference)
import jax
import jax.numpy as jnp
from jax.experimental import pallas as pl


def kernel(x_type, x_tok, x_small, edge_index, edge_type, batch, conv_weight, gru_wih, gru_whh, gru_bih, gru_bhh, ln_gamma, ln_beta, head_w1, head_b1, head_w2, head_b2):
    raise NotImplementedError("write your pallas kernel here")


# speedup vs baseline: 1.3346x; 1.3346x over previous
import functools

import jax
import jax.numpy as jnp
from jax import lax
from jax.experimental import pallas as pl
from jax.experimental.pallas import tpu as pltpu
from jax.experimental.pallas import tpu_sc as plsc

N = 100000
E = 1600000
NUM_TYPES = 29
DIM_TOK = 17
C = 48
BLOCKS = 2
TYPES = 3
STEPS = 2
G = 16

R = 10000
NB = N // R

NSUB = 16
NCORE = 2
ROWS = TYPES * N
NCHUNK = 14
CHUNK = 21760
ROWS_PAD = NCHUNK * CHUNK
TRASH = CHUNK
EPS = E // NSUB
SUB = 10000
NSUBP = EPS // SUB
STG = 2000
NSTG = SUB // STG
GB = 512
NGB = SUB // GB
DTILE = CHUNK // NSUB
ZR = 256


def _sc_scatter_kernel(m_hbm, src_hbm, dst_hbm, et_hbm, agg_hbm,
                       src_st, dst_st, et_st, gidx_b, sidx_b,
                       rows_v, zrows, acc_sh, sem):
    cid = lax.axis_index("c")
    sid = lax.axis_index("s")
    ebase0 = sid * EPS
    zero16 = jnp.zeros((16,), jnp.float32)
    i32_16 = jnp.zeros((16,), jnp.int32)

    def _z(r, _):
        for k in range(3):
            zrows[r, pl.ds(k * 16, 16)] = zero16
        return _
    lax.fori_loop(0, ZR, _z, 0)

    for ch in range(NCHUNK // NCORE):
        chunk_id = cid * (NCHUNK // NCORE) + ch
        lo = chunk_id * CHUNK
        hi = lo + CHUNK

        dbase = sid * DTILE
        for i in range(DTILE // ZR):
            pltpu.sync_copy(zrows, acc_sh.at[pl.ds(dbase + i * ZR, ZR)])
        rem = DTILE - (DTILE // ZR) * ZR
        if rem:
            pltpu.sync_copy(zrows.at[pl.ds(0, rem)],
                            acc_sh.at[pl.ds(dbase + (DTILE // ZR) * ZR, rem)])
        @pl.when(sid == 0)
        def _():
            pltpu.sync_copy(zrows.at[pl.ds(0, 16)],
                            acc_sh.at[pl.ds(CHUNK, 16)])
        plsc.subcore_barrier()

        def _subpass(sp, carry):
            ebase = ebase0 + sp * SUB

            def _stage(st, w):
                sbase = ebase + st * STG
                pltpu.sync_copy(src_hbm.at[pl.ds(sbase, STG)], src_st)
                pltpu.sync_copy(dst_hbm.at[pl.ds(sbase, STG)], dst_st)
                pltpu.sync_copy(et_hbm.at[pl.ds(sbase, STG)], et_st)

                def _vreg(v, w):
                    s = src_st[pl.ds(v * 16, 16)]
                    d = dst_st[pl.ds(v * 16, 16)]
                    t = et_st[pl.ds(v * 16, 16)]
                    sidx = t * N + d
                    msk = (sidx >= lo) & (sidx < hi)
                    gidx = t * N + s
                    k = msk.astype(jnp.int32)
                    _, gs = plsc.sort_key_val(k, gidx, descending=True)
                    _, ss = plsc.sort_key_val(k, sidx - lo, descending=True)
                    gidx_b[pl.ds(w, 16)] = gs
                    sidx_b[pl.ds(w, 16)] = ss
                    return w + jnp.sum(k)
                return lax.fori_loop(0, STG // 16, _vreg, w)

            w = lax.fori_loop(0, NSTG, _stage, jnp.int32(0))

            def _pad(j, c):
                gidx_b[pl.ds(w + j * 16, 16)] = i32_16
                sidx_b[pl.ds(w + j * 16, 16)] = i32_16 + TRASH
                return c
            lax.fori_loop(0, GB // 16, _pad, 0)

            for b in range(NGB):
                @pl.when(b * GB < w)
                def _():
                    pltpu.async_copy(m_hbm.at[gidx_b.at[pl.ds(b * GB, GB)]],
                                     rows_v, sem).wait()
                    pltpu.sync_copy(rows_v,
                                    acc_sh.at[sidx_b.at[pl.ds(b * GB, GB)]],
                                    add=True)
            return carry
        lax.fori_loop(0, NSUBP, _subpass, 0)
        plsc.subcore_barrier()

        pltpu.sync_copy(acc_sh.at[pl.ds(dbase, DTILE)],
                        agg_hbm.at[pl.ds(lo + dbase, DTILE)])
        plsc.subcore_barrier()


def _sc_scatter_add(m, src, dst, et):
    mesh = plsc.VectorSubcoreMesh(core_axis_name="c", subcore_axis_name="s")
    kern = functools.partial(
        pl.kernel, _sc_scatter_kernel, mesh=mesh,
        compiler_params=pltpu.CompilerParams(
            needs_layout_passes=False, use_tc_tiling_on_sc=False),
        out_type=jax.ShapeDtypeStruct((ROWS_PAD, C), jnp.float32),
        scratch_types=[
            pltpu.VMEM((STG,), jnp.int32),
            pltpu.VMEM((STG,), jnp.int32),
            pltpu.VMEM((STG,), jnp.int32),
            pltpu.VMEM((NGB * GB + GB + 16,), jnp.int32),
            pltpu.VMEM((NGB * GB + GB + 16,), jnp.int32),
            pltpu.VMEM((GB, C), jnp.float32),
            pltpu.VMEM((ZR, C), jnp.float32),
            pltpu.VMEM_SHARED((CHUNK + 16, C), jnp.float32),
            pltpu.SemaphoreType.DMA,
        ],
    )()
    return kern(m, src, dst, et)


def _feat_kernel(xt_ref, xk_ref, xs_ref, o_ref):
    xt = xt_ref[...].astype(jnp.int32)
    xk = jnp.clip(xk_ref[...], 0.0, DIM_TOK - 1.0).astype(jnp.int32)
    it = lax.broadcasted_iota(jnp.int32, (R, NUM_TYPES), 1)
    ik = lax.broadcasted_iota(jnp.int32, (R, DIM_TOK), 1)
    oh_t = (it == xt).astype(jnp.float32)
    oh_k = (ik == xk).astype(jnp.float32)
    o_ref[...] = jnp.concatenate([oh_t, oh_k, xs_ref[...]], axis=1)


def _build_h(x_type, x_tok, x_small):
    return pl.pallas_call(
        _feat_kernel,
        grid=(NB,),
        in_specs=[pl.BlockSpec((R, 1), lambda i: (i, 0)),
                  pl.BlockSpec((R, 1), lambda i: (i, 0)),
                  pl.BlockSpec((R, 2), lambda i: (i, 0))],
        out_specs=pl.BlockSpec((R, C), lambda i: (i, 0)),
        out_shape=jax.ShapeDtypeStruct((N, C), jnp.float32),
    )(x_type.astype(jnp.float32), x_tok.astype(jnp.float32), x_small)


def _msg_kernel(hh_ref, w_ref, o_ref):
    o_ref[...] = jnp.dot(hh_ref[0], w_ref[0],
                         preferred_element_type=jnp.float32)[None]


def _messages(hh, w3):
    return pl.pallas_call(
        _msg_kernel,
        grid=(TYPES, NB),
        in_specs=[pl.BlockSpec((1, R, C), lambda t, i: (t, i, 0)),
                  pl.BlockSpec((1, C, C), lambda t, i: (t, 0, 0))],
        out_specs=pl.BlockSpec((1, R, C), lambda t, i: (t, i, 0)),
        out_shape=jax.ShapeDtypeStruct((TYPES, N, C), jnp.float32),
    )(hh, w3)


def _gru_kernel(agg_ref, hh_ref, wih_ref, whh_ref, bih_ref, bhh_ref, o_ref):
    gi = jnp.dot(agg_ref[...], wih_ref[0],
                 preferred_element_type=jnp.float32) + bih_ref[0, 0][None]
    gh = jnp.dot(hh_ref[0], whh_ref[0],
                 preferred_element_type=jnp.float32) + bhh_ref[0, 0][None]
    r = jax.nn.sigmoid(gi[:, 0:C] + gh[:, 0:C])
    z = jax.nn.sigmoid(gi[:, C:2 * C] + gh[:, C:2 * C])
    n = jnp.tanh(gi[:, 2 * C:] + r * gh[:, 2 * C:])
    o_ref[...] = ((1.0 - z) * n + z * hh_ref[0])[None]


def _gru(agg, hh, wih_t, whh_t, bih, bhh):
    return pl.pallas_call(
        _gru_kernel,
        grid=(TYPES, NB),
        in_specs=[pl.BlockSpec((R, C), lambda t, i: (t * NB + i, 0)),
                  pl.BlockSpec((1, R, C), lambda t, i: (t, i, 0)),
                  pl.BlockSpec((1, C, 3 * C), lambda t, i: (t, 0, 0)),
                  pl.BlockSpec((1, C, 3 * C), lambda t, i: (t, 0, 0)),
                  pl.BlockSpec((1, 8, 3 * C), lambda t, i: (t, 0, 0)),
                  pl.BlockSpec((1, 8, 3 * C), lambda t, i: (t, 0, 0))],
        out_specs=pl.BlockSpec((1, R, C), lambda t, i: (t, i, 0)),
        out_shape=jax.ShapeDtypeStruct((TYPES, N, C), jnp.float32),
    )(agg, hh, wih_t, whh_t, bih, bhh)


def _comb_kernel(h_ref, hh_ref, g_ref, b_ref, o_ref):
    x2 = h_ref[...] + hh_ref[0] + hh_ref[1] + hh_ref[2]
    mu = jnp.mean(x2, axis=-1, keepdims=True)
    var = jnp.mean((x2 - mu) * (x2 - mu), axis=-1, keepdims=True)
    y = g_ref[0][None] * (x2 - mu) / jnp.sqrt(var + 1e-5) + b_ref[0][None]
    o_ref[...] = jnp.maximum(y, 0.0)


def _combine(h, hh, gamma8, beta8):
    return pl.pallas_call(
        _comb_kernel,
        grid=(NB,),
        in_specs=[pl.BlockSpec((R, C), lambda i: (i, 0)),
                  pl.BlockSpec((TYPES, R, C), lambda i: (0, i, 0)),
                  pl.BlockSpec((8, C), lambda i: (0, 0)),
                  pl.BlockSpec((8, C), lambda i: (0, 0))],
        out_specs=pl.BlockSpec((R, C), lambda i: (i, 0)),
        out_shape=jax.ShapeDtypeStruct((N, C), jnp.float32),
    )(h, hh, gamma8, beta8)


def _pool_kernel(h_ref, bf_ref, w1_ref, b1_ref, w2_ref, b2_ref, o_ref,
                 hg_acc, cnt_acc):
    i = pl.program_id(0)

    @pl.when(i == 0)
    def _():
        hg_acc[...] = jnp.zeros((G, C), jnp.float32)
        cnt_acc[...] = jnp.zeros((8, G), jnp.float32)

    ig = lax.broadcasted_iota(jnp.int32, (R, G), 1)
    oh = (ig == bf_ref[...].astype(jnp.int32)).astype(jnp.float32)
    hg_acc[...] += lax.dot_general(oh, h_ref[...],
                                   (((0,), (0,)), ((), ())),
                                   preferred_element_type=jnp.float32)
    cnt_acc[0, :] += jnp.sum(oh, axis=0)

    @pl.when(i == NB - 1)
    def _():
        cnt = jnp.maximum(cnt_acc[0, :], 1.0)
        hg = hg_acc[...] / cnt[:, None]
        z1 = jnp.maximum(jnp.dot(hg, w1_ref[...],
                                 preferred_element_type=jnp.float32)
                         + b1_ref[0][None], 0.0)
        o_ref[...] = jnp.dot(z1, w2_ref[...],
                             preferred_element_type=jnp.float32) \
            + b2_ref[0][None]


def _pool_head(h, batch_f, w1t, b1_8, w2t, b2_8):
    return pl.pallas_call(
        _pool_kernel,
        grid=(NB,),
        in_specs=[pl.BlockSpec((R, C), lambda i: (i, 0)),
                  pl.BlockSpec((R, 1), lambda i: (i, 0)),
                  pl.BlockSpec((C, C), lambda i: (0, 0)),
                  pl.BlockSpec((8, C), lambda i: (0, 0)),
                  pl.BlockSpec((C, 2), lambda i: (0, 0)),
                  pl.BlockSpec((8, 2), lambda i: (0, 0))],
        out_specs=pl.BlockSpec((G, 2), lambda i: (0, 0)),
        out_shape=jax.ShapeDtypeStruct((G, 2), jnp.float32),
        scratch_shapes=[pltpu.VMEM((G, C), jnp.float32),
                        pltpu.VMEM((8, G), jnp.float32)],
    )(h, batch_f, w1t, b1_8, w2t, b2_8)


def kernel(x_type, x_tok, x_small, edge_index, edge_type, batch,
           conv_weight, gru_wih, gru_whh, gru_bih, gru_bhh,
           ln_gamma, ln_beta, head_w1, head_b1, head_w2, head_b2):
    src = edge_index[0].astype(jnp.int32)
    dst = edge_index[1].astype(jnp.int32)
    et = edge_type.astype(jnp.int32)

    wih_t = jnp.transpose(gru_wih, (0, 1, 3, 2))
    whh_t = jnp.transpose(gru_whh, (0, 1, 3, 2))
    bih8 = jnp.broadcast_to(gru_bih[:, :, None, :], (BLOCKS, TYPES, 8, 3 * C))
    bhh8 = jnp.broadcast_to(gru_bhh[:, :, None, :], (BLOCKS, TYPES, 8, 3 * C))
    gamma8 = jnp.broadcast_to(ln_gamma[:, None, :], (BLOCKS, 8, C))
    beta8 = jnp.broadcast_to(ln_beta[:, None, :], (BLOCKS, 8, C))

    h = _build_h(x_type, x_tok, x_small)
    for b in range(BLOCKS):
        hh = jnp.broadcast_to(h[None], (TYPES, N, C)) + 0.0
        for s in range(STEPS):
            m = _messages(hh, conv_weight[b, :, s]).reshape(TYPES * N, C)
            agg = _sc_scatter_add(m, src, dst, et)
            hh = _gru(agg, hh, wih_t[b], whh_t[b], bih8[b], bhh8[b])
        h = _combine(h, hh, gamma8[b], beta8[b])

    return _pool_head(h, batch.astype(jnp.float32)[:, None],
                      jnp.transpose(head_w1), jnp.broadcast_to(head_b1[None], (8, C)),
                      jnp.transpose(head_w2), jnp.broadcast_to(head_b2[None], (8, 2)))


# speedup vs baseline: 5.9935x; 4.4908x over previous
import functools

import jax
import jax.numpy as jnp
from jax import lax
from jax.experimental import pallas as pl
from jax.experimental.pallas import tpu as pltpu
from jax.experimental.pallas import tpu_sc as plsc

N = 100000
E = 1600000
NUM_TYPES = 29
DIM_TOK = 17
C = 48
BLOCKS = 2
TYPES = 3
STEPS = 2
G = 16

R = 10000
NB = N // R

NSUB = 16
NCORE = 2
ROWS = TYPES * N
NCHUNK = 14
CHUNK = 21760
ROWS_PAD = NCHUNK * CHUNK
TRASH = CHUNK
EPS = E // NSUB
SUB = 10000
NSUBP = EPS // SUB
STG = 2000
NSTG = SUB // STG
GB = 512
NGB = SUB // GB
DTILE = CHUNK // NSUB
ZR = 256


CHPC = NCHUNK // NCORE
SUBP = 4000
NSUBP2 = EPS // SUBP
NSTG2 = SUBP // STG
CAP = 100864
NBKT = NCORE * NSUB * CHPC
NPB = CAP // GB
VCAP = SUBP + GB + 16


def _sc_prep_kernel(src_hbm, dst_hbm, et_hbm, gbuf_hbm, sbuf_hbm, cnt_hbm,
                    src_st, dst_st, et_st, gvb, svb, cnt_v):
    cid = lax.axis_index("c")
    sid = lax.axis_index("s")
    ebase0 = sid * EPS
    i32_16 = jnp.zeros((16,), jnp.int32)
    base_lo = cid * CHPC * CHUNK
    bid0 = (cid * NSUB + sid) * CHPC

    def _subpass(sp, curs):
        ebase = ebase0 + sp * SUBP

        def _stage(st, ws):
            sbase = ebase + st * STG
            pltpu.sync_copy(src_hbm.at[pl.ds(sbase, STG)], src_st)
            pltpu.sync_copy(dst_hbm.at[pl.ds(sbase, STG)], dst_st)
            pltpu.sync_copy(et_hbm.at[pl.ds(sbase, STG)], et_st)

            def _vreg(v, ws):
                s = src_st[pl.ds(v * 16, 16)]
                d = dst_st[pl.ds(v * 16, 16)]
                t = et_st[pl.ds(v * 16, 16)]
                sidx = t * N + d - base_lo
                gidx = t * N + s
                out = []
                for ch in range(CHPC):
                    w = ws[ch]
                    rel = sidx - ch * CHUNK
                    msk = (rel >= 0) & (rel < CHUNK)
                    k = msk.astype(jnp.int32)
                    _, gs = plsc.sort_key_val(k, gidx, descending=True)
                    _, rs = plsc.sort_key_val(k, rel, descending=True)
                    gvb[ch][pl.ds(w, 16)] = gs
                    svb[ch][pl.ds(w, 16)] = rs
                    out.append(w + jnp.sum(k))
                return tuple(out)
            return lax.fori_loop(0, STG // 16, _vreg, ws)

        ws = lax.fori_loop(0, NSTG2, _stage,
                           tuple(jnp.int32(0) for _ in range(CHPC)))

        new_curs = []
        for ch in range(CHPC):
            w = ws[ch]
            cur = curs[ch]

            def _pad(j, c):
                gvb[ch][pl.ds(w + j * 16, 16)] = i32_16
                svb[ch][pl.ds(w + j * 16, 16)] = i32_16 + TRASH
                return c
            lax.fori_loop(0, GB // 16, _pad, 0)
            hbase = pl.multiple_of((bid0 + ch) * CAP + cur, 8)
            for b in range(SUBP // GB):
                @pl.when(b * GB < w)
                def _():
                    pltpu.sync_copy(gvb[ch].at[pl.ds(b * GB, GB)],
                                    gbuf_hbm.at[pl.ds(hbase + b * GB, GB)])
                    pltpu.sync_copy(svb[ch].at[pl.ds(b * GB, GB)],
                                    sbuf_hbm.at[pl.ds(hbase + b * GB, GB)])
            new_curs.append(cur + ((w + 7) // 8) * 8)
        return tuple(new_curs)

    curs = lax.fori_loop(0, NSUBP2, _subpass,
                         tuple(jnp.int32(0) for _ in range(CHPC)))

    def _trash(j, c):
        gvb[0][pl.ds(j * 16, 16)] = i32_16
        svb[0][pl.ds(j * 16, 16)] = i32_16 + TRASH
        return c
    lax.fori_loop(0, GB // 16, _trash, 0)
    for ch in range(CHPC):
        hbase = pl.multiple_of((bid0 + ch) * CAP + curs[ch], 8)
        pltpu.sync_copy(gvb[0].at[pl.ds(0, GB)],
                        gbuf_hbm.at[pl.ds(hbase, GB)])
        pltpu.sync_copy(svb[0].at[pl.ds(0, GB)],
                        sbuf_hbm.at[pl.ds(hbase, GB)])
        cnt_v[...] = jnp.broadcast_to(curs[ch], (16,)).astype(jnp.int32)
        pltpu.sync_copy(cnt_v, cnt_hbm.at[pl.ds((bid0 + ch) * 16, 16)])


def _sc_prep(src, dst, et):
    mesh = plsc.VectorSubcoreMesh(core_axis_name="c", subcore_axis_name="s")
    kern = functools.partial(
        pl.kernel, _sc_prep_kernel, mesh=mesh,
        compiler_params=pltpu.CompilerParams(
            needs_layout_passes=False, use_tc_tiling_on_sc=False),
        out_type=(jax.ShapeDtypeStruct((NBKT * CAP,), jnp.int32),
                  jax.ShapeDtypeStruct((NBKT * CAP,), jnp.int32),
                  jax.ShapeDtypeStruct((NBKT * 16,), jnp.int32)),
        scratch_types=[
            pltpu.VMEM((STG,), jnp.int32),
            pltpu.VMEM((STG,), jnp.int32),
            pltpu.VMEM((STG,), jnp.int32),
            [pltpu.VMEM((VCAP,), jnp.int32) for _ in range(CHPC)],
            [pltpu.VMEM((VCAP,), jnp.int32) for _ in range(CHPC)],
            pltpu.VMEM((16,), jnp.int32),
        ],
    )()
    return kern(src, dst, et)


def _sc_pass_kernel(m_hbm, gbuf_hbm, sbuf_hbm, cnt_hbm, agg_hbm,
                    gidx_v, sidx_v, rows_v, zrows, cnt_v, acc_sh, sem):
    cid = lax.axis_index("c")
    sid = lax.axis_index("s")
    zero16 = jnp.zeros((16,), jnp.float32)
    bid0 = (cid * NSUB + sid) * CHPC

    def _z(r, c):
        for k in range(3):
            zrows[r, pl.ds(k * 16, 16)] = zero16
        return c
    lax.fori_loop(0, ZR, _z, 0)

    for ch in range(CHPC):
        lo = (cid * CHPC + ch) * CHUNK
        dbase = sid * DTILE
        for i in range(DTILE // ZR):
            pltpu.sync_copy(zrows, acc_sh.at[pl.ds(dbase + i * ZR, ZR)])
        rem = DTILE % ZR
        if rem:
            pltpu.sync_copy(zrows.at[pl.ds(0, rem)],
                            acc_sh.at[pl.ds(dbase + (DTILE // ZR) * ZR, rem)])
        @pl.when(sid == 0)
        def _():
            pltpu.sync_copy(zrows.at[pl.ds(0, 16)],
                            acc_sh.at[pl.ds(CHUNK, 16)])
        plsc.subcore_barrier()

        pltpu.sync_copy(cnt_hbm.at[pl.ds((bid0 + ch) * 16, 16)], cnt_v)
        cnt = jnp.sum(cnt_v[...]) // 16
        hbase = (bid0 + ch) * CAP

        def _batch(b, c):
            @pl.when(b * GB < cnt)
            def _():
                pltpu.sync_copy(gbuf_hbm.at[pl.ds(hbase + b * GB, GB)],
                                gidx_v)
                pltpu.sync_copy(sbuf_hbm.at[pl.ds(hbase + b * GB, GB)],
                                sidx_v)
                pltpu.async_copy(m_hbm.at[gidx_v], rows_v, sem).wait()
                pltpu.sync_copy(rows_v, acc_sh.at[sidx_v], add=True)
            return c
        lax.fori_loop(0, NPB, _batch, 0)
        plsc.subcore_barrier()

        pltpu.sync_copy(acc_sh.at[pl.ds(dbase, DTILE)],
                        agg_hbm.at[pl.ds(lo + dbase, DTILE)])
        plsc.subcore_barrier()


def _sc_pass(m, gbuf, sbuf, cnt):
    mesh = plsc.VectorSubcoreMesh(core_axis_name="c", subcore_axis_name="s")
    kern = functools.partial(
        pl.kernel, _sc_pass_kernel, mesh=mesh,
        compiler_params=pltpu.CompilerParams(
            needs_layout_passes=False, use_tc_tiling_on_sc=False),
        out_type=jax.ShapeDtypeStruct((ROWS_PAD, C), jnp.float32),
        scratch_types=[
            pltpu.VMEM((GB,), jnp.int32),
            pltpu.VMEM((GB,), jnp.int32),
            pltpu.VMEM((GB, C), jnp.float32),
            pltpu.VMEM((ZR, C), jnp.float32),
            pltpu.VMEM((16,), jnp.int32),
            pltpu.VMEM_SHARED((CHUNK + 16, C), jnp.float32),
            pltpu.SemaphoreType.DMA,
        ],
    )()
    return kern(m, gbuf, sbuf, cnt)


def _feat_kernel(xt_ref, xk_ref, xs_ref, o_ref):
    xt = xt_ref[...].astype(jnp.int32)
    xk = jnp.clip(xk_ref[...], 0.0, DIM_TOK - 1.0).astype(jnp.int32)
    it = lax.broadcasted_iota(jnp.int32, (R, NUM_TYPES), 1)
    ik = lax.broadcasted_iota(jnp.int32, (R, DIM_TOK), 1)
    oh_t = (it == xt).astype(jnp.float32)
    oh_k = (ik == xk).astype(jnp.float32)
    o_ref[...] = jnp.concatenate([oh_t, oh_k, xs_ref[...]], axis=1)


def _build_h(x_type, x_tok, x_small):
    return pl.pallas_call(
        _feat_kernel,
        grid=(NB,),
        in_specs=[pl.BlockSpec((R, 1), lambda i: (i, 0)),
                  pl.BlockSpec((R, 1), lambda i: (i, 0)),
                  pl.BlockSpec((R, 2), lambda i: (i, 0))],
        out_specs=pl.BlockSpec((R, C), lambda i: (i, 0)),
        out_shape=jax.ShapeDtypeStruct((N, C), jnp.float32),
    )(x_type.astype(jnp.float32), x_tok.astype(jnp.float32), x_small)


def _msg_kernel(hh_ref, w_ref, o_ref):
    o_ref[...] = jnp.dot(hh_ref[0], w_ref[0],
                         preferred_element_type=jnp.float32)[None]


def _messages(hh, w3):
    return pl.pallas_call(
        _msg_kernel,
        grid=(TYPES, NB),
        in_specs=[pl.BlockSpec((1, R, C), lambda t, i: (t, i, 0)),
                  pl.BlockSpec((1, C, C), lambda t, i: (t, 0, 0))],
        out_specs=pl.BlockSpec((1, R, C), lambda t, i: (t, i, 0)),
        out_shape=jax.ShapeDtypeStruct((TYPES, N, C), jnp.float32),
    )(hh, w3)


def _gru_kernel(agg_ref, hh_ref, wih_ref, whh_ref, bih_ref, bhh_ref, o_ref):
    gi = jnp.dot(agg_ref[...], wih_ref[0],
                 preferred_element_type=jnp.float32) + bih_ref[0, 0][None]
    gh = jnp.dot(hh_ref[0], whh_ref[0],
                 preferred_element_type=jnp.float32) + bhh_ref[0, 0][None]
    r = jax.nn.sigmoid(gi[:, 0:C] + gh[:, 0:C])
    z = jax.nn.sigmoid(gi[:, C:2 * C] + gh[:, C:2 * C])
    n = jnp.tanh(gi[:, 2 * C:] + r * gh[:, 2 * C:])
    o_ref[...] = ((1.0 - z) * n + z * hh_ref[0])[None]


def _gru(agg, hh, wih_t, whh_t, bih, bhh):
    return pl.pallas_call(
        _gru_kernel,
        grid=(TYPES, NB),
        in_specs=[pl.BlockSpec((R, C), lambda t, i: (t * NB + i, 0)),
                  pl.BlockSpec((1, R, C), lambda t, i: (t, i, 0)),
                  pl.BlockSpec((1, C, 3 * C), lambda t, i: (t, 0, 0)),
                  pl.BlockSpec((1, C, 3 * C), lambda t, i: (t, 0, 0)),
                  pl.BlockSpec((1, 8, 3 * C), lambda t, i: (t, 0, 0)),
                  pl.BlockSpec((1, 8, 3 * C), lambda t, i: (t, 0, 0))],
        out_specs=pl.BlockSpec((1, R, C), lambda t, i: (t, i, 0)),
        out_shape=jax.ShapeDtypeStruct((TYPES, N, C), jnp.float32),
    )(agg, hh, wih_t, whh_t, bih, bhh)


def _comb_kernel(h_ref, hh_ref, g_ref, b_ref, o_ref):
    x2 = h_ref[...] + hh_ref[0] + hh_ref[1] + hh_ref[2]
    mu = jnp.mean(x2, axis=-1, keepdims=True)
    var = jnp.mean((x2 - mu) * (x2 - mu), axis=-1, keepdims=True)
    y = g_ref[0][None] * (x2 - mu) / jnp.sqrt(var + 1e-5) + b_ref[0][None]
    o_ref[...] = jnp.maximum(y, 0.0)


def _combine(h, hh, gamma8, beta8):
    return pl.pallas_call(
        _comb_kernel,
        grid=(NB,),
        in_specs=[pl.BlockSpec((R, C), lambda i: (i, 0)),
                  pl.BlockSpec((TYPES, R, C), lambda i: (0, i, 0)),
                  pl.BlockSpec((8, C), lambda i: (0, 0)),
                  pl.BlockSpec((8, C), lambda i: (0, 0))],
        out_specs=pl.BlockSpec((R, C), lambda i: (i, 0)),
        out_shape=jax.ShapeDtypeStruct((N, C), jnp.float32),
    )(h, hh, gamma8, beta8)


def _pool_kernel(h_ref, bf_ref, w1_ref, b1_ref, w2_ref, b2_ref, o_ref,
                 hg_acc, cnt_acc):
    i = pl.program_id(0)

    @pl.when(i == 0)
    def _():
        hg_acc[...] = jnp.zeros((G, C), jnp.float32)
        cnt_acc[...] = jnp.zeros((8, G), jnp.float32)

    ig = lax.broadcasted_iota(jnp.int32, (R, G), 1)
    oh = (ig == bf_ref[...].astype(jnp.int32)).astype(jnp.float32)
    hg_acc[...] += lax.dot_general(oh, h_ref[...],
                                   (((0,), (0,)), ((), ())),
                                   preferred_element_type=jnp.float32)
    cnt_acc[0, :] += jnp.sum(oh, axis=0)

    @pl.when(i == NB - 1)
    def _():
        cnt = jnp.maximum(cnt_acc[0, :], 1.0)
        hg = hg_acc[...] / cnt[:, None]
        z1 = jnp.maximum(jnp.dot(hg, w1_ref[...],
                                 preferred_element_type=jnp.float32)
                         + b1_ref[0][None], 0.0)
        o_ref[...] = jnp.dot(z1, w2_ref[...],
                             preferred_element_type=jnp.float32) \
            + b2_ref[0][None]


def _pool_head(h, batch_f, w1t, b1_8, w2t, b2_8):
    return pl.pallas_call(
        _pool_kernel,
        grid=(NB,),
        in_specs=[pl.BlockSpec((R, C), lambda i: (i, 0)),
                  pl.BlockSpec((R, 1), lambda i: (i, 0)),
                  pl.BlockSpec((C, C), lambda i: (0, 0)),
                  pl.BlockSpec((8, C), lambda i: (0, 0)),
                  pl.BlockSpec((C, 2), lambda i: (0, 0)),
                  pl.BlockSpec((8, 2), lambda i: (0, 0))],
        out_specs=pl.BlockSpec((G, 2), lambda i: (0, 0)),
        out_shape=jax.ShapeDtypeStruct((G, 2), jnp.float32),
        scratch_shapes=[pltpu.VMEM((G, C), jnp.float32),
                        pltpu.VMEM((8, G), jnp.float32)],
    )(h, batch_f, w1t, b1_8, w2t, b2_8)


def kernel(x_type, x_tok, x_small, edge_index, edge_type, batch,
           conv_weight, gru_wih, gru_whh, gru_bih, gru_bhh,
           ln_gamma, ln_beta, head_w1, head_b1, head_w2, head_b2):
    src = edge_index[0].astype(jnp.int32)
    dst = edge_index[1].astype(jnp.int32)
    et = edge_type.astype(jnp.int32)

    wih_t = jnp.transpose(gru_wih, (0, 1, 3, 2))
    whh_t = jnp.transpose(gru_whh, (0, 1, 3, 2))
    bih8 = jnp.broadcast_to(gru_bih[:, :, None, :], (BLOCKS, TYPES, 8, 3 * C))
    bhh8 = jnp.broadcast_to(gru_bhh[:, :, None, :], (BLOCKS, TYPES, 8, 3 * C))
    gamma8 = jnp.broadcast_to(ln_gamma[:, None, :], (BLOCKS, 8, C))
    beta8 = jnp.broadcast_to(ln_beta[:, None, :], (BLOCKS, 8, C))

    h = _build_h(x_type, x_tok, x_small)
    gbuf, sbuf, cnt = _sc_prep(src, dst, et)
    for b in range(BLOCKS):
        hh = jnp.broadcast_to(h[None], (TYPES, N, C)) + 0.0
        for s in range(STEPS):
            m = _messages(hh, conv_weight[b, :, s]).reshape(TYPES * N, C)
            agg = _sc_pass(m, gbuf, sbuf, cnt)
            hh = _gru(agg, hh, wih_t[b], whh_t[b], bih8[b], bhh8[b])
        h = _combine(h, hh, gamma8[b], beta8[b])

    return _pool_head(h, batch.astype(jnp.float32)[:, None],
                      jnp.transpose(head_w1), jnp.broadcast_to(head_b1[None], (8, C)),
                      jnp.transpose(head_w2), jnp.broadcast_to(head_b2[None], (8, 2)))
